# baseline (speedup 1.0000x reference)
import functools

import jax
import jax.numpy as jnp
import numpy as np
from jax import lax
from jax.experimental import pallas as pl
from jax.experimental.pallas import tpu as pltpu
from jax.experimental.pallas import tpu_sc as plsc

_NC = 2
_NS = 16
_NW = _NC * _NS
_L = 16
_QW = 2048

_SIGN = np.int32(-2147483648)
_MASK = np.int32(2147483647)


@functools.partial(jax.jit, static_argnames=("f", "v", "d", "b"))
def _sc_gather(ids_bits, tab_t, *, f, v, d, b):
    units = f * d
    upw = units // _NW
    c0 = ((v // 2 + 127) // 128) * 128
    va = (v // 128) * 128
    vt = v - va
    c1 = va - c0
    ns = b // _QW
    grp = _QW // _L
    mesh = plsc.VectorSubcoreMesh(core_axis_name="c", subcore_axis_name="s")

    @functools.partial(
        pl.kernel,
        out_type=jax.ShapeDtypeStruct((f, d, b), jnp.float32),
        mesh=mesh,
        compiler_params=pltpu.CompilerParams(needs_layout_passes=False,
                                             use_tc_tiling_on_sc=True),
        scratch_types=[
            pltpu.VMEM((v,), jnp.float32),
            pltpu.VMEM((b,), jnp.float32),
            pltpu.VMEM((_QW,), jnp.float32),
            pltpu.VMEM((_QW,), jnp.float32),
            pltpu.VMEM((vt,), jnp.float32),
            pltpu.SemaphoreType.DMA,
            pltpu.SemaphoreType.DMA,
            pltpu.SemaphoreType.DMA,
            pltpu.SemaphoreType.DMA,
            pltpu.SemaphoreType.DMA,
        ],
    )
    def body(ids_hbm, tab_hbm, out_hbm, row, uni, p0, p1, tl, s0, s1, t0,
             t1, si):
        wid = lax.axis_index("s") * _NC + lax.axis_index("c")
        u0 = wid * upw
        f0 = u0 // d
        d0 = u0 % d

        pltpu.async_copy(ids_hbm.at[f0, :], uni, si)
        pltpu.async_copy(tab_hbm.at[f0, d0, pl.ds(0, c0)],
                         row.at[pl.ds(0, c0)], s0)
        pltpu.async_copy(tab_hbm.at[f0, d0, pl.ds(c0, c1)],
                         row.at[pl.ds(c0, c1)], s1)

        def unit_body(i, carry):
            u = u0 + i
            fi = u // d
            di = u % d
            un = u + 1
            fn = un // d
            dn = un % d
            more = i + 1 < upw

            pltpu.make_async_copy(ids_hbm.at[fi, :], uni, si).wait()
            pltpu.make_async_copy(tab_hbm.at[fi, di, pl.ds(0, c0)],
                                  row.at[pl.ds(0, c0)], s0).wait()

            def pass0(j, c):
                idx = plsc.bitcast(uni[pl.ds(j * _L, _L)], jnp.int32)
                g = plsc.load_gather(row, [jnp.minimum(idx, c0 - 1)])
                enc = plsc.bitcast(idx | _SIGN, jnp.float32)
                uni[pl.ds(j * _L, _L)] = jnp.where(idx < c0, g, enc)
                return c

            lax.fori_loop(0, b // _L, pass0, 0, unroll=8)

            @pl.when(more)
            def _():
                pltpu.async_copy(tab_hbm.at[fn, dn, pl.ds(0, c0)],
                                 row.at[pl.ds(0, c0)], s0)

            pltpu.make_async_copy(tab_hbm.at[fi, di, pl.ds(c0, c1)],
                                  row.at[pl.ds(c0, c1)], s1).wait()
            pltpu.sync_copy(tab_hbm.at[fi, di, pl.ds(va, vt)], tl)
            for k in range(vt // _L):
                row[pl.ds(va + k * _L, _L)] = tl[pl.ds(k * _L, _L)]

            for q in range(ns):
                pq, sq = (p0, t0) if q % 2 == 0 else (p1, t1)
                po = (q - 2 if q >= 2 else q + ns - 2) * _QW

                @pl.when(jnp.logical_or(i > 0, q >= 2))
                def _():
                    up = u if q >= 2 else u - 1
                    pltpu.make_async_copy(
                        pq, out_hbm.at[up // d, up % d, pl.ds(po, _QW)],
                        sq).wait()

                def pass1(j, c):
                    jj = q * grp + j
                    val = uni[pl.ds(jj * _L, _L)]
                    pi = plsc.bitcast(val, jnp.int32)
                    ai = jnp.minimum(pi & _MASK, v - 1)
                    g = plsc.load_gather(row, [ai])
                    pq[pl.ds(j * _L, _L)] = jnp.where(pi < 0, g, val)
                    return c

                lax.fori_loop(0, grp, pass1, 0, unroll=8)
                pltpu.async_copy(pq, out_hbm.at[fi, di, pl.ds(q * _QW, _QW)],
                                 sq)

            @pl.when(more)
            def _():
                pltpu.async_copy(tab_hbm.at[fn, dn, pl.ds(c0, c1)],
                                 row.at[pl.ds(c0, c1)], s1)
                pltpu.async_copy(ids_hbm.at[fn, :], uni, si)

            return carry

        lax.fori_loop(0, upw, unit_body, 0, unroll=False)

        ul = u0 + upw - 1
        fl = ul // d
        dl = ul % d
        for q in (ns - 2, ns - 1):
            pq, sq = (p0, t0) if q % 2 == 0 else (p1, t1)
            pltpu.make_async_copy(pq, out_hbm.at[fl, dl, pl.ds(q * _QW, _QW)],
                                  sq).wait()

    return body(ids_bits, tab_t)


def kernel(embedding_ids, tables):
    f, v, d = tables.shape
    b = embedding_ids.shape[0]
    ids_bits = lax.bitcast_convert_type(embedding_ids.T, jnp.float32)
    out_t = _sc_gather(ids_bits, tables.transpose(0, 2, 1),
                       f=f, v=v, d=d, b=b)
    return out_t.transpose(0, 2, 1)


# speedup vs baseline: 1.4339x; 1.4339x over previous
import functools

import jax
import jax.numpy as jnp
from jax import lax
from jax.experimental import pallas as pl
from jax.experimental.pallas import tpu as pltpu
from jax.experimental.pallas import tpu_sc as plsc

_NC = 2
_NS = 16
_NW = _NC * _NS
_L = 16
_QW = 4096


@functools.partial(jax.jit, static_argnames=("f", "v", "d", "b"))
def _sc_gather(ids_t, tab_t, *, f, v, d, b):
    units = f * d
    upw = units // _NW
    cw = ((v // 4 + 127) // 128) * 128
    va = (v // 128) * 128
    vt = v - va
    chunks = [(k * cw, min(cw, va - k * cw)) for k in range(4)]
    nq = b // _QW
    mesh = plsc.VectorSubcoreMesh(core_axis_name="c", subcore_axis_name="s")

    @functools.partial(
        pl.kernel,
        out_type=jax.ShapeDtypeStruct((f, d, b), jnp.float32),
        mesh=mesh,
        compiler_params=pltpu.CompilerParams(needs_layout_passes=False,
                                             use_tc_tiling_on_sc=True),
        scratch_types=[
            pltpu.VMEM((v,), jnp.float32),
            pltpu.VMEM((b,), jnp.int32),
            pltpu.VMEM((_QW,), jnp.float32),
            pltpu.VMEM((_QW,), jnp.float32),
            pltpu.VMEM((vt,), jnp.float32),
            pltpu.SemaphoreType.DMA,
            pltpu.SemaphoreType.DMA,
            pltpu.SemaphoreType.DMA,
            pltpu.SemaphoreType.DMA,
            pltpu.SemaphoreType.DMA,
            pltpu.SemaphoreType.DMA,
        ],
    )
    def body(ids_hbm, tab_hbm, out_hbm, row, ids, p0, p1, tl, s0, s1, s2,
             s3, t0, t1):
        wid = lax.axis_index("s") * _NC + lax.axis_index("c")
        u0 = wid * upw

        def unit_body(i, carry):
            u = u0 + i
            fi = u // d
            di = u % d

            @pl.when(jnp.logical_or(i == 0, di == 0))
            def _():
                pltpu.sync_copy(ids_hbm.at[fi, :], ids)

            sems = (s0, s1, s2, s3)
            for (c_off, c_len), sem in zip(chunks, sems):
                pltpu.async_copy(tab_hbm.at[fi, di, pl.ds(c_off, c_len)],
                                 row.at[pl.ds(c_off, c_len)], sem)
            pltpu.sync_copy(tab_hbm.at[fi, di, pl.ds(va, vt)], tl)
            for k in range(vt // _L):
                row[pl.ds(va + k * _L, _L)] = tl[pl.ds(k * _L, _L)]
            for (c_off, c_len), sem in zip(chunks, sems):
                pltpu.make_async_copy(tab_hbm.at[fi, di, pl.ds(c_off, c_len)],
                                      row.at[pl.ds(c_off, c_len)], sem).wait()

            for q in range(nq):
                pq, sq = (p0, t0) if q % 2 == 0 else (p1, t1)
                prev = q - 2 if q >= 2 else q + 2
                po = prev * _QW

                @pl.when(jnp.logical_or(i > 0, q >= 2))
                def _():
                    up = u - 1 if q < 2 else u
                    pltpu.make_async_copy(
                        pq, out_hbm.at[up // d, up % d, pl.ds(po, _QW)],
                        sq).wait()

                def grp_body(j, c):
                    idx = ids[pl.ds(q * _QW + j * _L, _L)]
                    pq[pl.ds(j * _L, _L)] = plsc.load_gather(row, [idx])
                    return c

                lax.fori_loop(0, _QW // _L, grp_body, 0, unroll=16)
                pltpu.async_copy(pq, out_hbm.at[fi, di, pl.ds(q * _QW, _QW)],
                                 sq)
            return carry

        lax.fori_loop(0, upw, unit_body, 0, unroll=False)

        ul = u0 + upw - 1
        fl = ul // d
        dl = ul % d
        for q, (pq, sq) in enumerate(((p0, t0), (p1, t1))):
            po = (nq - 2 + q) * _QW
            pltpu.make_async_copy(pq, out_hbm.at[fl, dl, pl.ds(po, _QW)],
                                  sq).wait()

    return body(ids_t, tab_t)


def kernel(embedding_ids, tables):
    f, v, d = tables.shape
    b = embedding_ids.shape[0]
    out_t = _sc_gather(embedding_ids.T, tables.transpose(0, 2, 1),
                       f=f, v=v, d=d, b=b)
    return out_t.transpose(0, 2, 1)


# speedup vs baseline: 1.4443x; 1.0072x over previous
import functools

import jax
import jax.numpy as jnp
from jax import lax
from jax.experimental import pallas as pl
from jax.experimental.pallas import tpu as pltpu
from jax.experimental.pallas import tpu_sc as plsc

_NC = 2
_NS = 16
_NW = _NC * _NS
_L = 16
_QW = 4096


@functools.partial(jax.jit, static_argnames=("f", "v", "d", "b"))
def _sc_gather(ids_t, tab_t, *, f, v, d, b):
    units = f * d
    upw = units // _NW
    cw = ((v // 8 + 127) // 128) * 128
    va = (v // 128) * 128
    vt = v - va
    chunks = [(k * cw, min(cw, va - k * cw)) for k in range(8)]
    nq = b // _QW
    mesh = plsc.VectorSubcoreMesh(core_axis_name="c", subcore_axis_name="s")

    @functools.partial(
        pl.kernel,
        out_type=jax.ShapeDtypeStruct((f, d, b), jnp.float32),
        mesh=mesh,
        compiler_params=pltpu.CompilerParams(needs_layout_passes=False,
                                             use_tc_tiling_on_sc=True),
        scratch_types=[
            pltpu.VMEM((v,), jnp.float32),
            pltpu.VMEM((b,), jnp.int32),
            pltpu.VMEM((_QW,), jnp.float32),
            pltpu.VMEM((_QW,), jnp.float32),
            pltpu.VMEM((vt,), jnp.float32),
            pltpu.SemaphoreType.DMA,
            pltpu.SemaphoreType.DMA,
            pltpu.SemaphoreType.DMA,
            pltpu.SemaphoreType.DMA,
            pltpu.SemaphoreType.DMA,
            pltpu.SemaphoreType.DMA,
            pltpu.SemaphoreType.DMA,
            pltpu.SemaphoreType.DMA,
            pltpu.SemaphoreType.DMA,
            pltpu.SemaphoreType.DMA,
        ],
    )
    def body(ids_hbm, tab_hbm, out_hbm, row, ids, p0, p1, tl, s0, s1, s2,
             s3, s4, s5, s6, s7, t0, t1):
        wid = lax.axis_index("s") * _NC + lax.axis_index("c")
        u0 = wid * upw

        def unit_body(i, carry):
            u = u0 + i
            fi = u // d
            di = u % d

            @pl.when(jnp.logical_or(i == 0, di == 0))
            def _():
                pltpu.sync_copy(ids_hbm.at[fi, :], ids)

            sems = (s0, s1, s2, s3, s4, s5, s6, s7)
            for (c_off, c_len), sem in zip(chunks, sems):
                pltpu.async_copy(tab_hbm.at[fi, di, pl.ds(c_off, c_len)],
                                 row.at[pl.ds(c_off, c_len)], sem)
            pltpu.sync_copy(tab_hbm.at[fi, di, pl.ds(va, vt)], tl)
            for k in range(vt // _L):
                row[pl.ds(va + k * _L, _L)] = tl[pl.ds(k * _L, _L)]
            for (c_off, c_len), sem in zip(chunks, sems):
                pltpu.make_async_copy(tab_hbm.at[fi, di, pl.ds(c_off, c_len)],
                                      row.at[pl.ds(c_off, c_len)], sem).wait()

            for q in range(nq):
                pq, sq = (p0, t0) if q % 2 == 0 else (p1, t1)
                prev = q - 2 if q >= 2 else q + 2
                po = prev * _QW

                @pl.when(jnp.logical_or(i > 0, q >= 2))
                def _():
                    up = u - 1 if q < 2 else u
                    pltpu.make_async_copy(
                        pq, out_hbm.at[up // d, up % d, pl.ds(po, _QW)],
                        sq).wait()

                def grp_body(j, c):
                    idx = ids[pl.ds(q * _QW + j * _L, _L)]
                    pq[pl.ds(j * _L, _L)] = plsc.load_gather(row, [idx])
                    return c

                lax.fori_loop(0, _QW // _L, grp_body, 0, unroll=16)
                pltpu.async_copy(pq, out_hbm.at[fi, di, pl.ds(q * _QW, _QW)],
                                 sq)
            return carry

        lax.fori_loop(0, upw, unit_body, 0, unroll=False)

        ul = u0 + upw - 1
        fl = ul // d
        dl = ul % d
        for q, (pq, sq) in enumerate(((p0, t0), (p1, t1))):
            po = (nq - 2 + q) * _QW
            pltpu.make_async_copy(pq, out_hbm.at[fl, dl, pl.ds(po, _QW)],
                                  sq).wait()

    return body(ids_t, tab_t)


def kernel(embedding_ids, tables):
    f, v, d = tables.shape
    b = embedding_ids.shape[0]
    out_t = _sc_gather(embedding_ids.T, tables.transpose(0, 2, 1),
                       f=f, v=v, d=d, b=b)
    return out_t.transpose(0, 2, 1)


# speedup vs baseline: 1.4486x; 1.0030x over previous
import functools

import jax
import jax.numpy as jnp
from jax import lax
from jax.experimental import pallas as pl
from jax.experimental.pallas import tpu as pltpu
from jax.experimental.pallas import tpu_sc as plsc

_NC = 2
_NS = 16
_NW = _NC * _NS
_L = 16
_QW = 4096


@functools.partial(jax.jit, static_argnames=("f", "v", "d", "b"))
def _sc_gather(ids_t, tab_t, *, f, v, d, b):
    units = f * d
    upw = units // _NW
    cw = ((v // 8 + 127) // 128) * 128
    va = (v // 128) * 128
    vt = v - va
    chunks = [(k * cw, min(cw, va - k * cw)) for k in range(8)]
    nq = b // _QW
    mesh = plsc.VectorSubcoreMesh(core_axis_name="c", subcore_axis_name="s")

    @functools.partial(
        pl.kernel,
        out_type=jax.ShapeDtypeStruct((f, d, b), jnp.float32),
        mesh=mesh,
        compiler_params=pltpu.CompilerParams(needs_layout_passes=False,
                                             use_tc_tiling_on_sc=True),
        scratch_types=[
            pltpu.VMEM((v,), jnp.float32),
            pltpu.VMEM((b,), jnp.int32),
            pltpu.VMEM((_QW,), jnp.float32),
            pltpu.VMEM((_QW,), jnp.float32),
            pltpu.VMEM((vt,), jnp.float32),
            pltpu.SemaphoreType.DMA,
            pltpu.SemaphoreType.DMA,
            pltpu.SemaphoreType.DMA,
            pltpu.SemaphoreType.DMA,
            pltpu.SemaphoreType.DMA,
            pltpu.SemaphoreType.DMA,
            pltpu.SemaphoreType.DMA,
            pltpu.SemaphoreType.DMA,
            pltpu.SemaphoreType.DMA,
            pltpu.SemaphoreType.DMA,
        ],
    )
    def body(ids_hbm, tab_hbm, out_hbm, row, ids, p0, p1, tl, s0, s1, s2,
             s3, s4, s5, s6, s7, t0, t1):
        wid = lax.axis_index("s") * _NC + lax.axis_index("c")
        u0 = wid * upw

        def unit_body(i, carry):
            u = u0 + i
            fi = u // d
            di = u % d

            sems = (s0, s1, s2, s3, s4, s5, s6, s7)
            for (c_off, c_len), sem in zip(chunks, sems):
                pltpu.async_copy(tab_hbm.at[fi, di, pl.ds(c_off, c_len)],
                                 row.at[pl.ds(c_off, c_len)], sem)

            @pl.when(jnp.logical_or(i == 0, di == 0))
            def _():
                pltpu.sync_copy(ids_hbm.at[fi, :], ids)

            pltpu.sync_copy(tab_hbm.at[fi, di, pl.ds(va, vt)], tl)
            for k in range(vt // _L):
                row[pl.ds(va + k * _L, _L)] = tl[pl.ds(k * _L, _L)]
            for (c_off, c_len), sem in zip(chunks, sems):
                pltpu.make_async_copy(tab_hbm.at[fi, di, pl.ds(c_off, c_len)],
                                      row.at[pl.ds(c_off, c_len)], sem).wait()

            for q in range(nq):
                pq, sq = (p0, t0) if q % 2 == 0 else (p1, t1)
                prev = q - 2 if q >= 2 else q + 2
                po = prev * _QW

                @pl.when(jnp.logical_or(i > 0, q >= 2))
                def _():
                    up = u - 1 if q < 2 else u
                    pltpu.make_async_copy(
                        pq, out_hbm.at[up // d, up % d, pl.ds(po, _QW)],
                        sq).wait()

                def grp_body(j, c):
                    idx = ids[pl.ds(q * _QW + j * _L, _L)]
                    pq[pl.ds(j * _L, _L)] = plsc.load_gather(row, [idx])
                    return c

                lax.fori_loop(0, _QW // _L, grp_body, 0, unroll=16)
                pltpu.async_copy(pq, out_hbm.at[fi, di, pl.ds(q * _QW, _QW)],
                                 sq)
            return carry

        lax.fori_loop(0, upw, unit_body, 0, unroll=False)

        ul = u0 + upw - 1
        fl = ul // d
        dl = ul % d
        for q, (pq, sq) in enumerate(((p0, t0), (p1, t1))):
            po = (nq - 2 + q) * _QW
            pltpu.make_async_copy(pq, out_hbm.at[fl, dl, pl.ds(po, _QW)],
                                  sq).wait()

    return body(ids_t, tab_t)


def kernel(embedding_ids, tables):
    f, v, d = tables.shape
    b = embedding_ids.shape[0]
    out_t = _sc_gather(embedding_ids.T, tables.transpose(0, 2, 1),
                       f=f, v=v, d=d, b=b)
    return out_t.transpose(0, 2, 1)


# speedup vs baseline: 1.4519x; 1.0023x over previous
import functools

import jax
import jax.numpy as jnp
from jax import lax
from jax.experimental import pallas as pl
from jax.experimental.pallas import tpu as pltpu
from jax.experimental.pallas import tpu_sc as plsc

_NC = 2
_NS = 16
_NW = _NC * _NS
_L = 16
_QW = 4096


@functools.partial(jax.jit, static_argnames=("f", "v", "d", "b"))
def _sc_gather(ids_t, tab_t, *, f, v, d, b):
    units = f * d
    upw = units // _NW
    cw = ((v // 8 + 127) // 128) * 128
    va = (v // 128) * 128
    vt = v - va
    chunks = [(k * cw, min(cw, va - k * cw)) for k in range(8)]
    nq = b // _QW
    mesh = plsc.VectorSubcoreMesh(core_axis_name="c", subcore_axis_name="s")

    @functools.partial(
        pl.kernel,
        out_type=jax.ShapeDtypeStruct((f, d, b), jnp.float32),
        mesh=mesh,
        compiler_params=pltpu.CompilerParams(needs_layout_passes=False,
                                             use_tc_tiling_on_sc=True),
        scratch_types=[
            pltpu.VMEM((v,), jnp.float32),
            pltpu.VMEM((b,), jnp.int32),
            pltpu.VMEM((_QW,), jnp.float32),
            pltpu.VMEM((_QW,), jnp.float32),
            pltpu.VMEM((vt,), jnp.float32),
            pltpu.SemaphoreType.DMA,
            pltpu.SemaphoreType.DMA,
            pltpu.SemaphoreType.DMA,
            pltpu.SemaphoreType.DMA,
            pltpu.SemaphoreType.DMA,
            pltpu.SemaphoreType.DMA,
            pltpu.SemaphoreType.DMA,
            pltpu.SemaphoreType.DMA,
            pltpu.SemaphoreType.DMA,
            pltpu.SemaphoreType.DMA,
        ],
    )
    def body(ids_hbm, tab_hbm, out_hbm, row, ids, p0, p1, tl, s0, s1, s2,
             s3, s4, s5, s6, s7, t0, t1):
        wid = lax.axis_index("s") * _NC + lax.axis_index("c")
        u0 = wid * upw

        def unit_body(i, carry):
            u = u0 + i
            fi = u // d
            di = u % d

            sems = (s0, s1, s2, s3, s4, s5, s6, s7)
            for (c_off, c_len), sem in zip(chunks, sems):
                pltpu.async_copy(tab_hbm.at[fi, di, pl.ds(c_off, c_len)],
                                 row.at[pl.ds(c_off, c_len)], sem)

            @pl.when(jnp.logical_or(i == 0, di == 0))
            def _():
                pltpu.sync_copy(ids_hbm.at[fi, :], ids)

            pltpu.sync_copy(tab_hbm.at[fi, di, pl.ds(va, vt)], tl)
            for k in range(vt // _L):
                row[pl.ds(va + k * _L, _L)] = tl[pl.ds(k * _L, _L)]
            for (c_off, c_len), sem in zip(chunks, sems):
                pltpu.make_async_copy(tab_hbm.at[fi, di, pl.ds(c_off, c_len)],
                                      row.at[pl.ds(c_off, c_len)], sem).wait()

            for q in range(nq):
                pq, sq = (p0, t0) if q % 2 == 0 else (p1, t1)
                prev = q - 2 if q >= 2 else q + 2
                po = prev * _QW

                @pl.when(jnp.logical_or(i > 0, q >= 2))
                def _():
                    up = u - 1 if q < 2 else u
                    pltpu.make_async_copy(
                        pq, out_hbm.at[up // d, up % d, pl.ds(po, _QW)],
                        sq).wait()

                def grp_body(j, c):
                    idx = ids[pl.ds(q * _QW + j * _L, _L)]
                    pq[pl.ds(j * _L, _L)] = plsc.load_gather(row, [idx])
                    return c

                lax.fori_loop(0, _QW // _L, grp_body, 0, unroll=32)
                pltpu.async_copy(pq, out_hbm.at[fi, di, pl.ds(q * _QW, _QW)],
                                 sq)
            return carry

        lax.fori_loop(0, upw, unit_body, 0, unroll=False)

        ul = u0 + upw - 1
        fl = ul // d
        dl = ul % d
        for q, (pq, sq) in enumerate(((p0, t0), (p1, t1))):
            po = (nq - 2 + q) * _QW
            pltpu.make_async_copy(pq, out_hbm.at[fl, dl, pl.ds(po, _QW)],
                                  sq).wait()

    return body(ids_t, tab_t)


def kernel(embedding_ids, tables):
    f, v, d = tables.shape
    b = embedding_ids.shape[0]
    out_t = _sc_gather(embedding_ids.T, tables.transpose(0, 2, 1),
                       f=f, v=v, d=d, b=b)
    return out_t.transpose(0, 2, 1)
